# baseline (speedup 1.0000x reference)
import functools

import jax
import jax.numpy as jnp
from jax.experimental import pallas as pl
from jax.experimental.pallas import tpu as pltpu

_ROWS = 128
_IN = 2048
_SAE = 32768
_K = 64
_BLK = 2048


def _encode_kernel(x_ref, w_ref, b_ref, out_ref):
    acc = jax.lax.dot_general(
        x_ref[...], w_ref[...],
        dimension_numbers=(((1,), (1,)), ((), ())),
        preferred_element_type=jnp.float32,
    )
    out_ref[...] = jnp.maximum(acc + b_ref[...], 0.0)


_NCH = 512
_CH = _SAE // _NCH


def _search_kth(bits, k):
    def body(i, t):
        cand = t | (jnp.int32(1) << (30 - i))
        cnt = jnp.sum((bits >= cand).astype(jnp.int32), axis=1, keepdims=True)
        return jnp.where(cnt >= k, cand, t)

    return jax.lax.fori_loop(0, 31, body,
                             jnp.zeros((bits.shape[0], 1), jnp.int32))


_TROWS = 16


def _thresh_kernel(pre_ref, t_ref):
    pa = pre_ref[...]
    bits = jax.lax.bitcast_convert_type(pa, jnp.int32)
    chunks = pa.reshape(_TROWS, _NCH, _CH)
    tops = []
    rem = chunks
    for _ in range(4):
        cm = jnp.max(rem, axis=2)
        tops.append(cm)
        rem = jnp.where(rem == cm[:, :, None], 0.0, rem)
    cand = jnp.concatenate(tops, axis=1)
    cbits = jax.lax.bitcast_convert_type(cand, jnp.int32)
    t = _search_kth(cbits, _K)
    cnt = jnp.sum((bits >= t).astype(jnp.int32), axis=1, keepdims=True)
    bad = cnt != _K
    t_ref[...] = jax.lax.bitcast_convert_type(t, jnp.float32)

    @pl.when(jnp.any(bad))
    def _():
        t_full = _search_kth(bits, _K)
        t_ref[...] = jax.lax.bitcast_convert_type(
            jnp.where(bad, t_full, t), jnp.float32)


def _decode_kernel(pre_ref, t_ref, wd_ref, bd_ref, sparse_ref, recon_ref,
                   acc_ref, *, nblk):
    i = pl.program_id(0)
    pa = pre_ref[...]
    s = jnp.where(pa >= t_ref[...], pa, 0.0)
    sparse_ref[...] = s
    part = jax.lax.dot_general(
        s, wd_ref[...],
        dimension_numbers=(((1,), (1,)), ((), ())),
        preferred_element_type=jnp.float32,
    )

    @pl.when(i == 0)
    def _():
        acc_ref[...] = part

    @pl.when(i > 0)
    def _():
        acc_ref[...] += part

    @pl.when(i == nblk - 1)
    def _():
        recon_ref[...] = acc_ref[...] + bd_ref[...]


def kernel(x, W_enc, b_enc, W_dec, b_dec):
    nblk = _SAE // _BLK
    b_enc2 = b_enc.reshape(1, _SAE)
    b_dec2 = b_dec.reshape(1, _IN)

    pre_act = pl.pallas_call(
        _encode_kernel,
        grid=(nblk,),
        in_specs=[
            pl.BlockSpec((_ROWS, _IN), lambda i: (0, 0)),
            pl.BlockSpec((_BLK, _IN), lambda i: (i, 0)),
            pl.BlockSpec((1, _BLK), lambda i: (0, i)),
        ],
        out_specs=pl.BlockSpec((_ROWS, _BLK), lambda i: (0, i)),
        out_shape=jax.ShapeDtypeStruct((_ROWS, _SAE), jnp.float32),
    )(x, W_enc, b_enc2)

    thresh = pl.pallas_call(
        _thresh_kernel,
        grid=(_ROWS // _TROWS,),
        in_specs=[pl.BlockSpec((_TROWS, _SAE), lambda i: (i, 0))],
        out_specs=pl.BlockSpec((_TROWS, 1), lambda i: (i, 0)),
        out_shape=jax.ShapeDtypeStruct((_ROWS, 1), jnp.float32),
    )(pre_act)

    sparse_act, recon = pl.pallas_call(
        functools.partial(_decode_kernel, nblk=nblk),
        grid=(nblk,),
        in_specs=[
            pl.BlockSpec((_ROWS, _BLK), lambda i: (0, i)),
            pl.BlockSpec((_ROWS, 1), lambda i: (0, 0)),
            pl.BlockSpec((_IN, _BLK), lambda i: (0, i)),
            pl.BlockSpec((1, _IN), lambda i: (0, 0)),
        ],
        out_specs=[
            pl.BlockSpec((_ROWS, _BLK), lambda i: (0, i)),
            pl.BlockSpec((_ROWS, _IN), lambda i: (0, 0)),
        ],
        out_shape=[
            jax.ShapeDtypeStruct((_ROWS, _SAE), jnp.float32),
            jax.ShapeDtypeStruct((_ROWS, _IN), jnp.float32),
        ],
        scratch_shapes=[pltpu.VMEM((_ROWS, _IN), jnp.float32)],
    )(pre_act, thresh, W_dec, b_dec2)

    return (recon, sparse_act)


# speedup vs baseline: 1.1043x; 1.1043x over previous
import functools

import jax
import jax.numpy as jnp
from jax.experimental import pallas as pl
from jax.experimental.pallas import tpu as pltpu

_ROWS = 128
_IN = 2048
_SAE = 32768
_K = 64
_BLK = 2048


def _encode_kernel(x_ref, w_ref, b_ref, out_ref):
    acc = jax.lax.dot_general(
        x_ref[...], w_ref[...],
        dimension_numbers=(((1,), (1,)), ((), ())),
        preferred_element_type=jnp.float32,
    )
    out_ref[...] = jnp.maximum(acc + b_ref[...], 0.0)


_NCH = 512
_CH = _SAE // _NCH


def _search_kth(bits, k):
    def body(i, t):
        cand = t | (jnp.int32(1) << (30 - i))
        cnt = jnp.sum((bits >= cand).astype(jnp.int32), axis=1, keepdims=True)
        return jnp.where(cnt >= k, cand, t)

    return jax.lax.fori_loop(0, 31, body,
                             jnp.zeros((bits.shape[0], 1), jnp.int32))


_TROWS = 16


def _thresh_kernel(pre_ref, t_ref):
    pa = pre_ref[...]
    bits = jax.lax.bitcast_convert_type(pa, jnp.int32)
    chunks = pa.reshape(_TROWS, _CH, _NCH)
    tops = []
    rem = chunks
    for _ in range(4):
        cm = jnp.max(rem, axis=1)
        tops.append(cm)
        rem = jnp.where(rem == cm[:, None, :], 0.0, rem)
    cand = jnp.concatenate(tops, axis=1)
    cbits = jax.lax.bitcast_convert_type(cand, jnp.int32)
    t = _search_kth(cbits, _K)
    cnt = jnp.sum((bits >= t).astype(jnp.int32), axis=1, keepdims=True)
    bad = cnt != _K
    t_ref[...] = jax.lax.bitcast_convert_type(t, jnp.float32)

    @pl.when(jnp.any(bad))
    def _():
        t_full = _search_kth(bits, _K)
        t_ref[...] = jax.lax.bitcast_convert_type(
            jnp.where(bad, t_full, t), jnp.float32)


def _decode_kernel(pre_ref, t_ref, wd_ref, bd_ref, sparse_ref, recon_ref,
                   acc_ref, *, nblk):
    i = pl.program_id(0)
    pa = pre_ref[...]
    s = jnp.where(pa >= t_ref[...], pa, 0.0)
    sparse_ref[...] = s
    part = jax.lax.dot_general(
        s, wd_ref[...],
        dimension_numbers=(((1,), (1,)), ((), ())),
        preferred_element_type=jnp.float32,
    )

    @pl.when(i == 0)
    def _():
        acc_ref[...] = part

    @pl.when(i > 0)
    def _():
        acc_ref[...] += part

    @pl.when(i == nblk - 1)
    def _():
        recon_ref[...] = acc_ref[...] + bd_ref[...]


def kernel(x, W_enc, b_enc, W_dec, b_dec):
    nblk = _SAE // _BLK
    b_enc2 = b_enc.reshape(1, _SAE)
    b_dec2 = b_dec.reshape(1, _IN)

    pre_act = pl.pallas_call(
        _encode_kernel,
        grid=(nblk,),
        in_specs=[
            pl.BlockSpec((_ROWS, _IN), lambda i: (0, 0)),
            pl.BlockSpec((_BLK, _IN), lambda i: (i, 0)),
            pl.BlockSpec((1, _BLK), lambda i: (0, i)),
        ],
        out_specs=pl.BlockSpec((_ROWS, _BLK), lambda i: (0, i)),
        out_shape=jax.ShapeDtypeStruct((_ROWS, _SAE), jnp.float32),
    )(x, W_enc, b_enc2)

    thresh = pl.pallas_call(
        _thresh_kernel,
        grid=(_ROWS // _TROWS,),
        in_specs=[pl.BlockSpec((_TROWS, _SAE), lambda i: (i, 0))],
        out_specs=pl.BlockSpec((_TROWS, 1), lambda i: (i, 0)),
        out_shape=jax.ShapeDtypeStruct((_ROWS, 1), jnp.float32),
    )(pre_act)

    sparse_act, recon = pl.pallas_call(
        functools.partial(_decode_kernel, nblk=nblk),
        grid=(nblk,),
        in_specs=[
            pl.BlockSpec((_ROWS, _BLK), lambda i: (0, i)),
            pl.BlockSpec((_ROWS, 1), lambda i: (0, 0)),
            pl.BlockSpec((_IN, _BLK), lambda i: (0, i)),
            pl.BlockSpec((1, _IN), lambda i: (0, 0)),
        ],
        out_specs=[
            pl.BlockSpec((_ROWS, _BLK), lambda i: (0, i)),
            pl.BlockSpec((_ROWS, _IN), lambda i: (0, 0)),
        ],
        out_shape=[
            jax.ShapeDtypeStruct((_ROWS, _SAE), jnp.float32),
            jax.ShapeDtypeStruct((_ROWS, _IN), jnp.float32),
        ],
        scratch_shapes=[pltpu.VMEM((_ROWS, _IN), jnp.float32)],
    )(pre_act, thresh, W_dec, b_dec2)

    return (recon, sparse_act)


# speedup vs baseline: 1.3100x; 1.1863x over previous
import functools

import jax
import jax.numpy as jnp
from jax.experimental import pallas as pl
from jax.experimental.pallas import tpu as pltpu

_ROWS = 128
_IN = 2048
_SAE = 32768
_K = 64
_BLK = 1024
_NBLK = _SAE // _BLK
_NCH = 512
_TOPC = 4


def _search_kth(bits, k):
    def body(i, t):
        cand = t | (jnp.int32(1) << (30 - i))
        cnt = jnp.sum((bits >= cand).astype(jnp.int32), axis=1, keepdims=True)
        return jnp.where(cnt >= k, cand, t)

    return jax.lax.fori_loop(0, 31, body,
                             jnp.zeros((bits.shape[0], 1), jnp.int32))


def _fused_kernel(x_ref, we_ref, be_ref, wd_ref, bd_ref,
                  sparse_ref, recon_ref,
                  pre_ref, top_ref, t_ref, acc_ref):
    i = pl.program_id(0)

    @pl.when(i == 0)
    def _():
        top_ref[...] = jnp.zeros_like(top_ref)

    @pl.when(i < _NBLK)
    def _encode():
        blk = jax.lax.dot_general(
            x_ref[...], we_ref[...],
            dimension_numbers=(((1,), (1,)), ((), ())),
            preferred_element_type=jnp.float32,
        )
        blk = jnp.maximum(blk + be_ref[...], 0.0)
        pre_ref[:, pl.ds(i * _BLK, _BLK)] = blk
        for g in range(_BLK // _NCH):
            v = blk[:, g * _NCH:(g + 1) * _NCH]
            for j in range(_TOPC):
                t = top_ref[:, j * _NCH:(j + 1) * _NCH]
                hi = jnp.maximum(t, v)
                v = jnp.minimum(t, v)
                top_ref[:, j * _NCH:(j + 1) * _NCH] = hi

    @pl.when(i >= _NBLK)
    def _decode():
        @pl.when(i == _NBLK)
        def _threshold():
            def count_ge(t):
                def body(b, cnt):
                    blk = pre_ref[:, pl.ds(b * _BLK, _BLK)]
                    bb = jax.lax.bitcast_convert_type(blk, jnp.int32)
                    return cnt + jnp.sum((bb >= t).astype(jnp.int32),
                                         axis=1, keepdims=True)
                return jax.lax.fori_loop(
                    0, _NBLK, body, jnp.zeros((_ROWS, 1), jnp.int32))

            cbits = jax.lax.bitcast_convert_type(top_ref[...], jnp.int32)
            t = _search_kth(cbits, _K)
            bad = count_ge(t) != _K
            t_ref[...] = jax.lax.bitcast_convert_type(t, jnp.float32)

            @pl.when(jnp.any(bad))
            def _():
                def body(p, tf):
                    cand = tf | (jnp.int32(1) << (30 - p))
                    return jnp.where(count_ge(cand) >= _K, cand, tf)
                t_full = jax.lax.fori_loop(
                    0, 31, body, jnp.zeros((_ROWS, 1), jnp.int32))
                t_ref[...] = jax.lax.bitcast_convert_type(
                    jnp.where(bad, t_full, t), jnp.float32)

        j = i - _NBLK
        pa = pre_ref[:, pl.ds(j * _BLK, _BLK)]
        s = jnp.where(pa >= t_ref[...], pa, 0.0)
        sparse_ref[...] = s
        part = jax.lax.dot_general(
            s, wd_ref[...],
            dimension_numbers=(((1,), (1,)), ((), ())),
            preferred_element_type=jnp.float32,
        )

        @pl.when(i == _NBLK)
        def _():
            acc_ref[...] = part

        @pl.when(i > _NBLK)
        def _():
            acc_ref[...] += part

        @pl.when(i == 2 * _NBLK - 1)
        def _():
            recon_ref[...] = acc_ref[...] + bd_ref[...]


def kernel(x, W_enc, b_enc, W_dec, b_dec):
    b_enc2 = b_enc.reshape(1, _SAE)
    b_dec2 = b_dec.reshape(1, _IN)

    sparse_act, recon = pl.pallas_call(
        _fused_kernel,
        grid=(2 * _NBLK,),
        in_specs=[
            pl.BlockSpec((_ROWS, _IN), lambda i: (0, 0)),
            pl.BlockSpec((_BLK, _IN), lambda i: (jnp.minimum(i, _NBLK - 1), 0)),
            pl.BlockSpec((1, _BLK), lambda i: (0, jnp.minimum(i, _NBLK - 1))),
            pl.BlockSpec((_IN, _BLK),
                         lambda i: (0, jnp.maximum(i, _NBLK) - _NBLK)),
            pl.BlockSpec((1, _IN), lambda i: (0, 0)),
        ],
        out_specs=[
            pl.BlockSpec((_ROWS, _BLK),
                         lambda i: (0, jnp.maximum(i, _NBLK) - _NBLK)),
            pl.BlockSpec((_ROWS, _IN), lambda i: (0, 0)),
        ],
        out_shape=[
            jax.ShapeDtypeStruct((_ROWS, _SAE), jnp.float32),
            jax.ShapeDtypeStruct((_ROWS, _IN), jnp.float32),
        ],
        scratch_shapes=[
            pltpu.VMEM((_ROWS, _SAE), jnp.float32),
            pltpu.VMEM((_ROWS, _TOPC * _NCH), jnp.float32),
            pltpu.VMEM((_ROWS, 1), jnp.float32),
            pltpu.VMEM((_ROWS, _IN), jnp.float32),
        ],
    )(x, W_enc, b_enc2, W_dec, b_dec2)

    return (recon, sparse_act)
